# baseline (speedup 1.0000x reference)
import jax
import jax.numpy as jnp
from jax import lax
from jax.experimental import pallas as pl
from jax.experimental.pallas import tpu as pltpu

_B, _S, _C = 32, 2048, 1024
_S_BLK = 512
_S_CHUNKS = _S // _S_BLK


def _precision_kernel(pred_ref, label_ref, out_ref, counts_ref):
    b = pl.program_id(0)
    s = pl.program_id(1)

    x = pred_ref[0]
    lane = lax.broadcasted_iota(jnp.int32, (_S_BLK, _C), 1)
    m = jnp.max(x, axis=1, keepdims=True)
    first = jnp.min(jnp.where(x == m, lane, _C), axis=1, keepdims=True)
    onehot = (first == lane).astype(jnp.int32)
    partial = jnp.sum(onehot, axis=0, keepdims=True)

    @pl.when(s == 0)
    def _():
        counts_ref[pl.ds(b, 1), :] = partial

    @pl.when(s != 0)
    def _():
        counts_ref[pl.ds(b, 1), :] = counts_ref[pl.ds(b, 1), :] + partial

    @pl.when((b == _B - 1) & (s == _S_CHUNKS - 1))
    def _():
        counts = counts_ref[...]
        label = label_ref[...]
        lane2 = lax.broadcasted_iota(jnp.int32, (_B, _C), 1)
        nonzero_cls = lane2 >= 1
        kd = dict(axis=(0, 1), keepdims=True)
        total_char = jnp.sum(jnp.where(nonzero_cls, label, 0), **kd)
        fn = jnp.sum(jnp.where(nonzero_cls, jnp.maximum(label - counts, 0), 0), **kd)
        zero_pred = jnp.sum(jnp.where(lane2 == 0, counts, 0), **kd)
        total_pred = (_B * _S - zero_pred).astype(jnp.float32)
        correct = (total_char - fn).astype(jnp.float32)
        out_ref[...] = correct / (total_pred + 1e-6)


def kernel(pred, label):
    out = pl.pallas_call(
        _precision_kernel,
        grid=(_B, _S_CHUNKS),
        in_specs=[
            pl.BlockSpec((1, _S_BLK, _C), lambda b, s: (b, s, 0)),
            pl.BlockSpec((_B, _C), lambda b, s: (0, 0)),
        ],
        out_specs=pl.BlockSpec((1, 1), lambda b, s: (0, 0)),
        out_shape=jax.ShapeDtypeStruct((1, 1), jnp.float32),
        scratch_shapes=[pltpu.VMEM((_B, _C), jnp.int32)],
    )(pred, label)
    return out[0, 0]


# speedup vs baseline: 1.3023x; 1.3023x over previous
import jax
import jax.numpy as jnp
from jax import lax
from jax.experimental import pallas as pl
from jax.experimental.pallas import tpu as pltpu

_B, _S, _C = 32, 2048, 1024
_S_BLK = 1024
_S_CHUNKS = _S // _S_BLK


def _precision_kernel(pred_ref, label_ref, out_ref, counts_ref):
    b = pl.program_id(0)
    s = pl.program_id(1)

    x = pred_ref[0]
    lane = lax.broadcasted_iota(jnp.int32, (_S_BLK, _C), 1)
    m = jnp.max(x, axis=1, keepdims=True)
    first = jnp.min(jnp.where(x == m, lane, _C), axis=1, keepdims=True)
    onehot = (first == lane).astype(jnp.int32)
    partial = jnp.sum(onehot, axis=0, keepdims=True)

    @pl.when(s == 0)
    def _():
        counts_ref[pl.ds(b, 1), :] = partial

    @pl.when(s != 0)
    def _():
        counts_ref[pl.ds(b, 1), :] = counts_ref[pl.ds(b, 1), :] + partial

    @pl.when((b == _B - 1) & (s == _S_CHUNKS - 1))
    def _():
        counts = counts_ref[...]
        label = label_ref[...]
        lane2 = lax.broadcasted_iota(jnp.int32, (_B, _C), 1)
        nonzero_cls = lane2 >= 1
        kd = dict(axis=(0, 1), keepdims=True)
        total_char = jnp.sum(jnp.where(nonzero_cls, label, 0), **kd)
        fn = jnp.sum(jnp.where(nonzero_cls, jnp.maximum(label - counts, 0), 0), **kd)
        zero_pred = jnp.sum(jnp.where(lane2 == 0, counts, 0), **kd)
        total_pred = (_B * _S - zero_pred).astype(jnp.float32)
        correct = (total_char - fn).astype(jnp.float32)
        out_ref[...] = correct / (total_pred + 1e-6)


def kernel(pred, label):
    out = pl.pallas_call(
        _precision_kernel,
        grid=(_B, _S_CHUNKS),
        in_specs=[
            pl.BlockSpec((1, _S_BLK, _C), lambda b, s: (b, s, 0)),
            pl.BlockSpec((_B, _C), lambda b, s: (0, 0)),
        ],
        out_specs=pl.BlockSpec((1, 1), lambda b, s: (0, 0)),
        out_shape=jax.ShapeDtypeStruct((1, 1), jnp.float32),
        scratch_shapes=[pltpu.VMEM((_B, _C), jnp.int32)],
    )(pred, label)
    return out[0, 0]


# speedup vs baseline: 1.5150x; 1.1633x over previous
import jax
import jax.numpy as jnp
from jax import lax
from jax.experimental import pallas as pl
from jax.experimental.pallas import tpu as pltpu

_B, _S, _C = 32, 2048, 1024
_S_BLK = 2048
_S_CHUNKS = _S // _S_BLK


def _precision_kernel(pred_ref, label_ref, out_ref, counts_ref):
    b = pl.program_id(0)
    s = pl.program_id(1)

    x = pred_ref[0]
    lane = lax.broadcasted_iota(jnp.int32, (_S_BLK, _C), 1)
    m = jnp.max(x, axis=1, keepdims=True)
    first = jnp.min(jnp.where(x == m, lane, _C), axis=1, keepdims=True)
    onehot = (first == lane).astype(jnp.int32)
    partial = jnp.sum(onehot, axis=0, keepdims=True)

    @pl.when(s == 0)
    def _():
        counts_ref[pl.ds(b, 1), :] = partial

    @pl.when(s != 0)
    def _():
        counts_ref[pl.ds(b, 1), :] = counts_ref[pl.ds(b, 1), :] + partial

    @pl.when((b == _B - 1) & (s == _S_CHUNKS - 1))
    def _():
        counts = counts_ref[...]
        label = label_ref[...]
        lane2 = lax.broadcasted_iota(jnp.int32, (_B, _C), 1)
        nonzero_cls = lane2 >= 1
        kd = dict(axis=(0, 1), keepdims=True)
        total_char = jnp.sum(jnp.where(nonzero_cls, label, 0), **kd)
        fn = jnp.sum(jnp.where(nonzero_cls, jnp.maximum(label - counts, 0), 0), **kd)
        zero_pred = jnp.sum(jnp.where(lane2 == 0, counts, 0), **kd)
        total_pred = (_B * _S - zero_pred).astype(jnp.float32)
        correct = (total_char - fn).astype(jnp.float32)
        out_ref[...] = correct / (total_pred + 1e-6)


def kernel(pred, label):
    out = pl.pallas_call(
        _precision_kernel,
        grid=(_B, _S_CHUNKS),
        in_specs=[
            pl.BlockSpec((1, _S_BLK, _C), lambda b, s: (b, s, 0)),
            pl.BlockSpec((_B, _C), lambda b, s: (0, 0)),
        ],
        out_specs=pl.BlockSpec((1, 1), lambda b, s: (0, 0)),
        out_shape=jax.ShapeDtypeStruct((1, 1), jnp.float32),
        scratch_shapes=[pltpu.VMEM((_B, _C), jnp.int32)],
    )(pred, label)
    return out[0, 0]


# speedup vs baseline: 1.6399x; 1.0824x over previous
import jax
import jax.numpy as jnp
from jax import lax
from jax.experimental import pallas as pl
from jax.experimental.pallas import tpu as pltpu

_B, _S, _C = 32, 2048, 1024
_B_BLK = 2
_B_CHUNKS = _B // _B_BLK


def _precision_kernel(pred_ref, label_ref, out_ref, counts_ref):
    b = pl.program_id(0)

    x = pred_ref[...].reshape(_B_BLK * _S, _C)
    lane = lax.broadcasted_iota(jnp.int32, (_B_BLK * _S, _C), 1)
    m = jnp.max(x, axis=1, keepdims=True)
    first = jnp.min(jnp.where(x == m, lane, _C), axis=1, keepdims=True)
    onehot = (first == lane).astype(jnp.int32)
    partial = onehot.reshape(_B_BLK, _S, _C).sum(axis=1)
    counts_ref[b] = partial

    @pl.when(b == _B_CHUNKS - 1)
    def _():
        counts = counts_ref[...].reshape(_B, _C)
        label = label_ref[...]
        lane2 = lax.broadcasted_iota(jnp.int32, (_B, _C), 1)
        nonzero_cls = lane2 >= 1
        kd = dict(axis=(0, 1), keepdims=True)
        total_char = jnp.sum(jnp.where(nonzero_cls, label, 0), **kd)
        fn = jnp.sum(jnp.where(nonzero_cls, jnp.maximum(label - counts, 0), 0), **kd)
        zero_pred = jnp.sum(jnp.where(lane2 == 0, counts, 0), **kd)
        total_pred = (_B * _S - zero_pred).astype(jnp.float32)
        correct = (total_char - fn).astype(jnp.float32)
        out_ref[...] = correct / (total_pred + 1e-6)


def kernel(pred, label):
    out = pl.pallas_call(
        _precision_kernel,
        grid=(_B_CHUNKS,),
        in_specs=[
            pl.BlockSpec((_B_BLK, _S, _C), lambda b: (b, 0, 0)),
            pl.BlockSpec((_B, _C), lambda b: (0, 0)),
        ],
        out_specs=pl.BlockSpec((1, 1), lambda b: (0, 0)),
        out_shape=jax.ShapeDtypeStruct((1, 1), jnp.float32),
        scratch_shapes=[pltpu.VMEM((_B_CHUNKS, _B_BLK, _C), jnp.int32)],
    )(pred, label)
    return out[0, 0]


# speedup vs baseline: 1.7564x; 1.0711x over previous
import jax
import jax.numpy as jnp
from jax import lax
from jax.experimental import pallas as pl
from jax.experimental.pallas import tpu as pltpu

_B, _S, _C = 32, 2048, 1024
_B_BLK = 2
_B_CHUNKS = _B // _B_BLK


def _precision_kernel(pred_ref, label_ref, out_ref, counts_ref):
    b = pl.program_id(0)

    x = pred_ref[...]
    m = jnp.max(x, axis=2, keepdims=True)
    counts_ref[b] = jnp.sum((x == m).astype(jnp.int32), axis=1)
    n_max = jnp.sum((x == m).astype(jnp.int32), axis=2)
    has_tie = jnp.max(n_max) > 1

    @pl.when(has_tie)
    def _():
        lane = lax.broadcasted_iota(jnp.int32, (_B_BLK, _S, _C), 2)
        first = jnp.min(jnp.where(x == m, lane, _C), axis=2, keepdims=True)
        onehot = (first == lane).astype(jnp.int32)
        counts_ref[b] = jnp.sum(onehot, axis=1)

    @pl.when(b == _B_CHUNKS - 1)
    def _():
        counts = counts_ref[...].reshape(_B, _C)
        label = label_ref[...]
        lane2 = lax.broadcasted_iota(jnp.int32, (_B, _C), 1)
        nonzero_cls = lane2 >= 1
        kd = dict(axis=(0, 1), keepdims=True)
        total_char = jnp.sum(jnp.where(nonzero_cls, label, 0), **kd)
        fn = jnp.sum(jnp.where(nonzero_cls, jnp.maximum(label - counts, 0), 0), **kd)
        zero_pred = jnp.sum(jnp.where(lane2 == 0, counts, 0), **kd)
        total_pred = (_B * _S - zero_pred).astype(jnp.float32)
        correct = (total_char - fn).astype(jnp.float32)
        out_ref[...] = correct / (total_pred + 1e-6)


def kernel(pred, label):
    out = pl.pallas_call(
        _precision_kernel,
        grid=(_B_CHUNKS,),
        in_specs=[
            pl.BlockSpec((_B_BLK, _S, _C), lambda b: (b, 0, 0)),
            pl.BlockSpec((_B, _C), lambda b: (0, 0)),
        ],
        out_specs=pl.BlockSpec((1, 1), lambda b: (0, 0)),
        out_shape=jax.ShapeDtypeStruct((1, 1), jnp.float32),
        scratch_shapes=[pltpu.VMEM((_B_CHUNKS, _B_BLK, _C), jnp.int32)],
    )(pred, label)
    return out[0, 0]


# speedup vs baseline: 1.8103x; 1.0307x over previous
import jax
import jax.numpy as jnp
from jax import lax
from jax.experimental import pallas as pl
from jax.experimental.pallas import tpu as pltpu

_B, _S, _C = 32, 2048, 1024
_B_BLK = 2
_B_CHUNKS = _B // _B_BLK


def _precision_kernel(pred_ref, label_ref, out_ref, counts_ref):
    b = pl.program_id(0)

    x = pred_ref[...]
    m = jnp.max(x, axis=2, keepdims=True)
    partial = jnp.sum((x == m).astype(jnp.int32), axis=1)
    counts_ref[b] = partial
    has_tie = jnp.sum(partial) != _B_BLK * _S

    @pl.when(has_tie)
    def _():
        lane = lax.broadcasted_iota(jnp.int32, (_B_BLK, _S, _C), 2)
        first = jnp.min(jnp.where(x == m, lane, _C), axis=2, keepdims=True)
        onehot = (first == lane).astype(jnp.int32)
        counts_ref[b] = jnp.sum(onehot, axis=1)

    @pl.when(b == _B_CHUNKS - 1)
    def _():
        counts = counts_ref[...].reshape(_B, _C)
        label = label_ref[...]
        lane2 = lax.broadcasted_iota(jnp.int32, (_B, _C), 1)
        nonzero_cls = lane2 >= 1
        kd = dict(axis=(0, 1), keepdims=True)
        total_char = jnp.sum(jnp.where(nonzero_cls, label, 0), **kd)
        fn = jnp.sum(jnp.where(nonzero_cls, jnp.maximum(label - counts, 0), 0), **kd)
        zero_pred = jnp.sum(jnp.where(lane2 == 0, counts, 0), **kd)
        total_pred = (_B * _S - zero_pred).astype(jnp.float32)
        correct = (total_char - fn).astype(jnp.float32)
        out_ref[...] = correct / (total_pred + 1e-6)


def kernel(pred, label):
    out = pl.pallas_call(
        _precision_kernel,
        grid=(_B_CHUNKS,),
        in_specs=[
            pl.BlockSpec((_B_BLK, _S, _C), lambda b: (b, 0, 0)),
            pl.BlockSpec((_B, _C), lambda b: (0, 0)),
        ],
        out_specs=pl.BlockSpec((1, 1), lambda b: (0, 0)),
        out_shape=jax.ShapeDtypeStruct((1, 1), jnp.float32),
        scratch_shapes=[pltpu.VMEM((_B_CHUNKS, _B_BLK, _C), jnp.int32)],
    )(pred, label)
    return out[0, 0]


# speedup vs baseline: 1.8417x; 1.0173x over previous
import jax
import jax.numpy as jnp
from jax import lax
from jax.experimental import pallas as pl
from jax.experimental.pallas import tpu as pltpu

_B, _S, _C = 32, 2048, 1024
_B_BLK = 2
_B_CHUNKS = _B // _B_BLK


def _count_max_hits(x):
    m = jnp.max(x, axis=2, keepdims=True)
    partial = jnp.sum((x == m).astype(jnp.int32), axis=1)
    has_tie = jnp.sum(partial) != _S

    def exact():
        lane = lax.broadcasted_iota(jnp.int32, (1, _S, _C), 2)
        first = jnp.min(jnp.where(x == m, lane, _C), axis=2, keepdims=True)
        onehot = (first == lane).astype(jnp.int32)
        return jnp.sum(onehot, axis=1)

    return jax.lax.cond(has_tie, exact, lambda: partial)


def _precision_kernel(pred0_ref, pred1_ref, label_ref, out_ref, counts_ref):
    b = pl.program_id(0)

    p0 = _count_max_hits(pred0_ref[...])
    p1 = _count_max_hits(pred1_ref[...])
    counts_ref[b] = jnp.concatenate([p0, p1], axis=0)

    @pl.when(b == _B_CHUNKS - 1)
    def _():
        counts = counts_ref[...].reshape(_B, _C)
        label = label_ref[...]
        lane2 = lax.broadcasted_iota(jnp.int32, (_B, _C), 1)
        nonzero_cls = lane2 >= 1
        kd = dict(axis=(0, 1), keepdims=True)
        total_char = jnp.sum(jnp.where(nonzero_cls, label, 0), **kd)
        fn = jnp.sum(jnp.where(nonzero_cls, jnp.maximum(label - counts, 0), 0), **kd)
        zero_pred = jnp.sum(jnp.where(lane2 == 0, counts, 0), **kd)
        total_pred = (_B * _S - zero_pred).astype(jnp.float32)
        correct = (total_char - fn).astype(jnp.float32)
        out_ref[...] = correct / (total_pred + 1e-6)


def kernel(pred, label):
    out = pl.pallas_call(
        _precision_kernel,
        grid=(_B_CHUNKS,),
        in_specs=[
            pl.BlockSpec((1, _S, _C), lambda b: (2 * b, 0, 0)),
            pl.BlockSpec((1, _S, _C), lambda b: (2 * b + 1, 0, 0)),
            pl.BlockSpec((_B, _C), lambda b: (0, 0)),
        ],
        out_specs=pl.BlockSpec((1, 1), lambda b: (0, 0)),
        out_shape=jax.ShapeDtypeStruct((1, 1), jnp.float32),
        scratch_shapes=[pltpu.VMEM((_B_CHUNKS, _B_BLK, _C), jnp.int32)],
    )(pred, pred, label)
    return out[0, 0]


# speedup vs baseline: 1.8475x; 1.0032x over previous
import jax
import jax.numpy as jnp
from jax import lax
from jax.experimental import pallas as pl
from jax.experimental.pallas import tpu as pltpu

_B, _S, _C = 32, 2048, 1024
_B_BLK = 2
_B_CHUNKS = _B // _B_BLK


def _count_max_hits(x):
    m = jnp.max(x, axis=2, keepdims=True)
    partial = jnp.sum((x == m).astype(jnp.int32), axis=1)
    has_tie = jnp.sum(partial) != _S // 2

    def exact():
        lane = lax.broadcasted_iota(jnp.int32, (1, _S // 2, _C), 2)
        first = jnp.min(jnp.where(x == m, lane, _C), axis=2, keepdims=True)
        onehot = (first == lane).astype(jnp.int32)
        return jnp.sum(onehot, axis=1)

    return jax.lax.cond(has_tie, exact, lambda: partial)


def _precision_kernel(p00_ref, p01_ref, p10_ref, p11_ref, label_ref, out_ref,
                      counts_ref):
    b = pl.program_id(0)

    p0 = _count_max_hits(p00_ref[...]) + _count_max_hits(p01_ref[...])
    p1 = _count_max_hits(p10_ref[...]) + _count_max_hits(p11_ref[...])
    counts_ref[b] = jnp.concatenate([p0, p1], axis=0)

    @pl.when(b == _B_CHUNKS - 1)
    def _():
        counts = counts_ref[...].reshape(_B, _C)
        label = label_ref[...]
        lane2 = lax.broadcasted_iota(jnp.int32, (_B, _C), 1)
        nonzero_cls = lane2 >= 1
        kd = dict(axis=(0, 1), keepdims=True)
        total_char = jnp.sum(jnp.where(nonzero_cls, label, 0), **kd)
        fn = jnp.sum(jnp.where(nonzero_cls, jnp.maximum(label - counts, 0), 0), **kd)
        zero_pred = jnp.sum(jnp.where(lane2 == 0, counts, 0), **kd)
        total_pred = (_B * _S - zero_pred).astype(jnp.float32)
        correct = (total_char - fn).astype(jnp.float32)
        out_ref[...] = correct / (total_pred + 1e-6)


def kernel(pred, label):
    out = pl.pallas_call(
        _precision_kernel,
        grid=(_B_CHUNKS,),
        in_specs=[
            pl.BlockSpec((1, _S // 2, _C), lambda b: (2 * b, 0, 0)),
            pl.BlockSpec((1, _S // 2, _C), lambda b: (2 * b, 1, 0)),
            pl.BlockSpec((1, _S // 2, _C), lambda b: (2 * b + 1, 0, 0)),
            pl.BlockSpec((1, _S // 2, _C), lambda b: (2 * b + 1, 1, 0)),
            pl.BlockSpec((_B, _C), lambda b: (0, 0)),
        ],
        out_specs=pl.BlockSpec((1, 1), lambda b: (0, 0)),
        out_shape=jax.ShapeDtypeStruct((1, 1), jnp.float32),
        scratch_shapes=[pltpu.VMEM((_B_CHUNKS, _B_BLK, _C), jnp.int32)],
    )(pred, pred, pred, pred, label)
    return out[0, 0]
